# Initial kernel scaffold; baseline (speedup 1.0000x reference)
import jax
import jax.numpy as jnp
from jax.experimental import pallas as pl


def kernel(table, x):
    raise NotImplementedError("write your pallas kernel here")


# speedup vs baseline: 2.2168x; 2.2168x over previous
import functools

import jax
import jax.numpy as jnp
from jax import lax
from jax.experimental import pallas as pl
from jax.experimental.pallas import tpu as pltpu
from jax.experimental.pallas import tpu_sc as plsc

VOCAB = 1000
EMBED = 300
EMBED_PAD = 384
BATCH = 1024
HIST = 200

B_TOTAL = BATCH * HIST
NUM_CORES = 2
NUM_SUBCORES = 16
NW = NUM_CORES * NUM_SUBCORES
B_PER_W = B_TOTAL // NW
CHUNK = 128
N_CHUNKS = B_PER_W // CHUNK
NBUF = 2


def _gather_body(table_hbm, idx_hbm, out_hbm, idx_c, rows_v, sem0, sem1):
    del sem1
    wid = lax.axis_index("s") * NUM_CORES + lax.axis_index("c")
    base = wid * B_PER_W

    def body(g, carry):
        c0 = base + g * CHUNK
        pltpu.sync_copy(idx_hbm.at[pl.ds(c0, CHUNK)], idx_c)
        pltpu.async_copy(table_hbm.at[idx_c], rows_v, sem0).wait()
        pltpu.sync_copy(rows_v, out_hbm.at[pl.ds(c0, CHUNK)])
        return carry

    lax.fori_loop(0, N_CHUNKS, body, 0)


@jax.jit
def _sc_gather(table_pad, idx_flat):
    k = functools.partial(
        pl.kernel,
        out_type=jax.ShapeDtypeStruct((B_TOTAL, EMBED_PAD), jnp.float32),
        mesh=plsc.VectorSubcoreMesh(core_axis_name="c", subcore_axis_name="s"),
        scratch_types=[
            pltpu.VMEM((CHUNK,), jnp.int32),
            pltpu.VMEM((CHUNK, EMBED_PAD), jnp.float32),
            pltpu.SemaphoreType.DMA,
            pltpu.SemaphoreType.DMA,
        ],
    )(_gather_body)
    return k(table_pad, idx_flat)


def kernel(table, x):
    idx_flat = x.reshape(B_TOTAL)
    table_pad = jnp.pad(table, ((0, 0), (0, EMBED_PAD - EMBED)))
    out = _sc_gather(table_pad, idx_flat)
    return out[:, :EMBED].reshape(BATCH, HIST, EMBED)


# speedup vs baseline: 2.3208x; 1.0469x over previous
import functools

import jax
import jax.numpy as jnp
from jax import lax
from jax.experimental import pallas as pl
from jax.experimental.pallas import tpu as pltpu
from jax.experimental.pallas import tpu_sc as plsc

VOCAB = 1000
EMBED = 300
EMBED_PAD = 384
BATCH = 1024
HIST = 200

B_TOTAL = BATCH * HIST
NUM_CORES = 2
NUM_SUBCORES = 16
NW = NUM_CORES * NUM_SUBCORES
B_PER_W = B_TOTAL // NW
CHUNK = 128
N_CHUNKS = B_PER_W // CHUNK
NBUF = 2


def _gather_body(table_hbm, idx_hbm, out_hbm, idx_v, rows_v, gsem, wsem0, wsem1):
    wsems = (wsem0, wsem1)
    wid = lax.axis_index("s") * NUM_CORES + lax.axis_index("c")
    base = wid * B_PER_W

    pltpu.sync_copy(idx_hbm.at[pl.ds(base, B_PER_W)], idx_v)

    def outer(i, carry):
        for b in range(NBUF):
            g = i * NBUF + b

            @pl.when(i >= 1)
            def _():
                pltpu.make_async_copy(
                    rows_v.at[b], out_hbm.at[pl.ds(base, CHUNK)], wsems[b]
                ).wait()

            pltpu.async_copy(
                table_hbm.at[idx_v.at[pl.ds(g * CHUNK, CHUNK)]],
                rows_v.at[b],
                gsem,
            ).wait()

            pltpu.async_copy(
                rows_v.at[b], out_hbm.at[pl.ds(base + g * CHUNK, CHUNK)], wsems[b]
            )
        return carry

    lax.fori_loop(0, N_CHUNKS // NBUF, outer, 0)

    for b in range(NBUF):
        pltpu.make_async_copy(
            rows_v.at[b], out_hbm.at[pl.ds(base, CHUNK)], wsems[b]
        ).wait()


@jax.jit
def _sc_gather(table_pad, idx_flat):
    k = functools.partial(
        pl.kernel,
        out_type=jax.ShapeDtypeStruct((B_TOTAL, EMBED_PAD), jnp.float32),
        mesh=plsc.VectorSubcoreMesh(core_axis_name="c", subcore_axis_name="s"),
        scratch_types=[
            pltpu.VMEM((B_PER_W,), jnp.int32),
            pltpu.VMEM((NBUF, CHUNK, EMBED_PAD), jnp.float32),
            pltpu.SemaphoreType.DMA,
            pltpu.SemaphoreType.DMA,
            pltpu.SemaphoreType.DMA,
        ],
    )(_gather_body)
    return k(table_pad, idx_flat)


def kernel(table, x):
    idx_flat = x.reshape(B_TOTAL)
    table_pad = jnp.pad(table, ((0, 0), (0, EMBED_PAD - EMBED)))
    out = _sc_gather(table_pad, idx_flat)
    return out[:, :EMBED].reshape(BATCH, HIST, EMBED)
